# Initial kernel scaffold; baseline (speedup 1.0000x reference)
import jax
import jax.numpy as jnp
from jax.experimental import pallas as pl


def kernel(feature, edge_index, W1, b1, W2, b2, gamma, beta):
    raise NotImplementedError("write your pallas kernel here")


# speedup vs baseline: 6.1381x; 6.1381x over previous
import functools

import jax
import jax.numpy as jnp
from jax import lax
from jax.experimental import pallas as pl
from jax.experimental.pallas import tpu as pltpu
from jax.experimental.pallas import tpu_sc as plsc

_NC = 2
_NS = 16
_CH = 128


def _sc_aggregate(feature, edge_index, zrow):
    N, D = feature.shape
    E = edge_index.shape[1]
    NW = _NC * _NS
    n_chunks = E // _CH
    rows_per_sub = ((N + _CH * _NS - 1) // (_CH * _NS)) * _CH
    NPAD = rows_per_sub * _NS
    outer = (n_chunks + NW - 1) // NW

    mesh = plsc.VectorSubcoreMesh(core_axis_name="c", subcore_axis_name="s")

    @functools.partial(
        pl.kernel,
        out_type=jax.ShapeDtypeStruct((_NC, NPAD, D), jnp.float32),
        mesh=mesh,
        scratch_types=[
            pltpu.VMEM((1, _CH), jnp.int32),
            pltpu.VMEM((1, _CH), jnp.int32),
            pltpu.VMEM((_CH, D), jnp.float32),
            pltpu.VMEM_SHARED((NPAD, D), jnp.float32),
        ],
    )
    def agg_kernel(feat_hbm, edge_hbm, zrow_hbm, out_hbm, sidx, didx, rows, acc):
        c = lax.axis_index("c")
        s = lax.axis_index("s")
        w = c * _NS + s

        pltpu.sync_copy(zrow_hbm, rows)

        @pl.loop(0, rows_per_sub // _CH)
        def _(j):
            pltpu.sync_copy(
                rows, acc.at[pl.ds(s * rows_per_sub + j * _CH, _CH), :])

        plsc.subcore_barrier()

        @pl.loop(0, outer)
        def _(j):
            chunk = w + j * NW

            @pl.when(chunk < n_chunks)
            def _():
                off = chunk * _CH
                pltpu.sync_copy(edge_hbm.at[0, pl.ds(off, _CH)], sidx.at[0])
                pltpu.sync_copy(edge_hbm.at[1, pl.ds(off, _CH)], didx.at[0])
                pltpu.sync_copy(feat_hbm.at[sidx.at[0]], rows)
                pltpu.sync_copy(rows, acc.at[didx.at[0]], add=True)

        plsc.subcore_barrier()

        pltpu.sync_copy(
            acc.at[pl.ds(s * rows_per_sub, rows_per_sub), :],
            out_hbm.at[c, pl.ds(s * rows_per_sub, rows_per_sub), :])

    return agg_kernel(feature, edge_index, zrow)


def _tc_fused(feature, partials, W1t, b1, W2t, b2, gamma, beta):
    N, D = feature.shape

    def body(f_ref, p_ref, w1_ref, b1_ref, w2_ref, b2_ref, g_ref, be_ref,
             o_ref):
        x = f_ref[...] + p_ref[0, pl.ds(0, N), :] + p_ref[1, pl.ds(0, N), :]
        h = jnp.dot(x, w1_ref[...], preferred_element_type=jnp.float32,
                    precision=lax.Precision.HIGHEST) + b1_ref[...]
        h = jnp.maximum(h, 0.0)
        h = jnp.dot(h, w2_ref[...], preferred_element_type=jnp.float32,
                    precision=lax.Precision.HIGHEST) + b2_ref[...]
        h = jnp.maximum(h, 0.0)
        mean = jnp.mean(h, axis=0, keepdims=True)
        var = jnp.mean(h * h, axis=0, keepdims=True) - mean * mean
        inv = lax.rsqrt(var + 1e-5)
        o_ref[...] = (h - mean) * inv * g_ref[...] + be_ref[...]

    return pl.pallas_call(
        body,
        out_shape=jax.ShapeDtypeStruct((N, D), jnp.float32),
    )(feature, partials, W1t, b1, W2t, b2, gamma, beta)


def kernel(feature, edge_index, W1, b1, W2, b2, gamma, beta):
    D = feature.shape[1]
    zrow = jnp.zeros((_CH, D), jnp.float32)
    partials = _sc_aggregate(feature, edge_index, zrow)
    return _tc_fused(feature, partials, W1.T, b1.reshape(1, D), W2.T,
                     b2.reshape(1, D), gamma.reshape(1, D),
                     beta.reshape(1, D))
